# baseline (device time: 395672 ns/iter reference)
import jax
import jax.numpy as jnp
from jax import lax
from jax.experimental import pallas as pl
from jax.experimental.pallas import tpu as pltpu

N_DEV = 4
M_CH = 2048
N_COL = 4096
N_HALF = N_COL // 2
SUB = 2
R = M_CH // SUB


def kernel(x, w_mat):
    xb = x.astype(jnp.bfloat16)
    wb = w_mat.astype(jnp.bfloat16)

    def body(x_ref, w_ref, out_ref, recv_hbm, x_tile, send_buf,
             recv_stage, out_stage, copy_sems, send_sems, recv_sems):
        my = lax.axis_index("i")
        right = lax.rem(my + 1, N_DEV)
        left = lax.rem(my + N_DEV - 1, N_DEV)

        barrier = pltpu.get_barrier_semaphore()
        for nbr in (left, right):
            pl.semaphore_signal(barrier, inc=1, device_id=(nbr,),
                                device_id_type=pl.DeviceIdType.MESH)
        pl.semaphore_wait(barrier, 2)

        tgt = (right, left)
        col0 = (0, N_HALF)

        def chunk_idx(dirn, s):
            if dirn == 0:
                return lax.rem(my + 2 * N_DEV - 1 - s, N_DEV)
            return lax.rem(my + 1 + s, N_DEV)

        def load_x_tile(c, t):
            cp = pltpu.make_async_copy(
                x_ref.at[pl.ds(c * M_CH + t * R, R), :], x_tile,
                copy_sems.at[0])
            cp.start()
            cp.wait()

        def gemm(dirn):
            return jnp.dot(
                x_tile[:, :],
                w_ref[:, col0[dirn]:col0[dirn] + N_HALF],
                preferred_element_type=jnp.float32,
            )

        def make_rdma(dirn, s, t):
            return pltpu.make_async_remote_copy(
                src_ref=send_buf.at[dirn, t],
                dst_ref=recv_hbm.at[dirn, s, t],
                send_sem=send_sems.at[dirn, s, t],
                recv_sem=recv_sems.at[dirn, s, t],
                device_id=(tgt[dirn],),
                device_id_type=pl.DeviceIdType.MESH,
            )

        def hop0(t, carry):
            for dirn in range(2):
                load_x_tile(chunk_idx(dirn, 0), t)
                send_buf[dirn, t, :, :] = gemm(dirn).astype(jnp.bfloat16)
                make_rdma(dirn, 0, t).start()
            return carry

        lax.fori_loop(0, SUB, hop0, 0)

        def hop(s, carry):
            for t in range(SUB):
                for dirn in range(2):
                    load_x_tile(chunk_idx(dirn, s), t)
                    make_rdma(dirn, s - 1, t).wait()
                    cp_r = pltpu.make_async_copy(
                        recv_hbm.at[dirn, s - 1, t], recv_stage,
                        copy_sems.at[1])
                    cp_r.start()
                    cp_r.wait()
                    send_buf[dirn, t, :, :] = (
                        recv_stage[:, :].astype(jnp.float32) + gemm(dirn)
                    ).astype(jnp.bfloat16)
                    make_rdma(dirn, s, t).start()
            return carry

        lax.fori_loop(1, N_DEV - 1, hop, 0)

        def final(t, carry):
            for dirn in range(2):
                if dirn == 0:
                    load_x_tile(my, t)
                make_rdma(dirn, N_DEV - 2, t).wait()
                cp_r = pltpu.make_async_copy(
                    recv_hbm.at[dirn, N_DEV - 2, t], recv_stage,
                    copy_sems.at[1])
                cp_r.start()
                cp_r.wait()
                out_stage[:, :] = jnp.maximum(
                    recv_stage[:, :].astype(jnp.float32) + gemm(dirn), 0.0)
                cp_o = pltpu.make_async_copy(
                    out_stage,
                    out_ref.at[pl.ds(t * R, R),
                               pl.ds(col0[dirn], N_HALF)],
                    copy_sems.at[2])
                cp_o.start()
                cp_o.wait()
            return carry

        lax.fori_loop(0, SUB, final, 0)

    out, _ = pl.pallas_call(
        body,
        out_shape=(
            jax.ShapeDtypeStruct((M_CH, N_COL), jnp.float32),
            jax.ShapeDtypeStruct((2, N_DEV - 1, SUB, R, N_HALF),
                                 jnp.bfloat16),
        ),
        in_specs=[
            pl.BlockSpec(memory_space=pl.ANY),
            pl.BlockSpec(memory_space=pltpu.MemorySpace.VMEM),
        ],
        out_specs=(
            pl.BlockSpec(memory_space=pl.ANY),
            pl.BlockSpec(memory_space=pl.ANY),
        ),
        scratch_shapes=[
            pltpu.MemorySpace.VMEM((R, 2048), jnp.bfloat16),
            pltpu.MemorySpace.VMEM((2, SUB, R, N_HALF), jnp.bfloat16),
            pltpu.MemorySpace.VMEM((R, N_HALF), jnp.bfloat16),
            pltpu.MemorySpace.VMEM((R, N_HALF), jnp.float32),
            pltpu.SemaphoreType.DMA((3,)),
            pltpu.SemaphoreType.DMA((2, N_DEV - 1, SUB)),
            pltpu.SemaphoreType.DMA((2, N_DEV - 1, SUB)),
        ],
        compiler_params=pltpu.CompilerParams(
            collective_id=0,
            vmem_limit_bytes=60 * 1024 * 1024,
        ),
    )(xb, wb)
    return out


# device time: 344998 ns/iter; 1.1469x vs baseline; 1.1469x over previous
import jax
import jax.numpy as jnp
from jax import lax
from jax.experimental import pallas as pl
from jax.experimental.pallas import tpu as pltpu

N_DEV = 4
M_CH = 2048
N_COL = 4096
N_HALF = N_COL // 2
SUB = 4
R = M_CH // SUB


def kernel(x, w_mat):
    wb = w_mat.astype(jnp.bfloat16)

    def body(x_ref, w_ref, out_ref, recv_hbm, x_tile, send_buf,
             recv_stage, out_stage, copy_sems, send_sems, recv_sems):
        my = lax.axis_index("i")
        right = lax.rem(my + 1, N_DEV)
        left = lax.rem(my + N_DEV - 1, N_DEV)

        barrier = pltpu.get_barrier_semaphore()
        for nbr in (left, right):
            pl.semaphore_signal(barrier, inc=1, device_id=(nbr,),
                                device_id_type=pl.DeviceIdType.MESH)
        pl.semaphore_wait(barrier, 2)

        tgt = (right, left)
        col0 = (0, N_HALF)

        def chunk_idx(dirn, s):
            if dirn == 0:
                return lax.rem(my + 2 * N_DEV - 1 - s, N_DEV)
            return lax.rem(my + 1 + s, N_DEV)

        def load_x_tile(c, t):
            cp = pltpu.make_async_copy(
                x_ref.at[pl.ds(c * M_CH + t * R, R), :], x_tile,
                copy_sems.at[0])
            cp.start()
            cp.wait()

        def gemm(dirn):
            return jnp.dot(
                x_tile[:, :].astype(jnp.bfloat16),
                w_ref[:, col0[dirn]:col0[dirn] + N_HALF],
                preferred_element_type=jnp.float32,
            )

        def make_rdma(dirn, s, t):
            return pltpu.make_async_remote_copy(
                src_ref=send_buf.at[dirn, t],
                dst_ref=recv_hbm.at[dirn, s, t],
                send_sem=send_sems.at[dirn, s, t],
                recv_sem=recv_sems.at[dirn, s, t],
                device_id=(tgt[dirn],),
                device_id_type=pl.DeviceIdType.MESH,
            )

        def hop0(t, carry):
            for dirn in range(2):
                load_x_tile(chunk_idx(dirn, 0), t)
                send_buf[dirn, t, :, :] = gemm(dirn).astype(jnp.bfloat16)
                make_rdma(dirn, 0, t).start()
            return carry

        lax.fori_loop(0, SUB, hop0, 0)

        def hop(s, carry):
            for t in range(SUB):
                for dirn in range(2):
                    load_x_tile(chunk_idx(dirn, s), t)
                    make_rdma(dirn, s - 1, t).wait()
                    cp_r = pltpu.make_async_copy(
                        recv_hbm.at[dirn, s - 1, t], recv_stage,
                        copy_sems.at[1])
                    cp_r.start()
                    cp_r.wait()
                    send_buf[dirn, t, :, :] = (
                        recv_stage[:, :].astype(jnp.float32) + gemm(dirn)
                    ).astype(jnp.bfloat16)
                    make_rdma(dirn, s, t).start()
            return carry

        lax.fori_loop(1, N_DEV - 1, hop, 0)

        def final(t, carry):
            for dirn in range(2):
                if dirn == 0:
                    load_x_tile(my, t)
                make_rdma(dirn, N_DEV - 2, t).wait()
                cp_r = pltpu.make_async_copy(
                    recv_hbm.at[dirn, N_DEV - 2, t], recv_stage,
                    copy_sems.at[1])
                cp_r.start()
                cp_r.wait()
                out_stage[:, :] = jnp.maximum(
                    recv_stage[:, :].astype(jnp.float32) + gemm(dirn), 0.0)
                cp_o = pltpu.make_async_copy(
                    out_stage,
                    out_ref.at[pl.ds(t * R, R),
                               pl.ds(col0[dirn], N_HALF)],
                    copy_sems.at[2])
                cp_o.start()
                cp_o.wait()
            return carry

        lax.fori_loop(0, SUB, final, 0)

    out, _ = pl.pallas_call(
        body,
        out_shape=(
            jax.ShapeDtypeStruct((M_CH, N_COL), jnp.float32),
            jax.ShapeDtypeStruct((2, N_DEV - 1, SUB, R, N_HALF),
                                 jnp.bfloat16),
        ),
        in_specs=[
            pl.BlockSpec(memory_space=pl.ANY),
            pl.BlockSpec(memory_space=pltpu.MemorySpace.VMEM),
        ],
        out_specs=(
            pl.BlockSpec(memory_space=pl.ANY),
            pl.BlockSpec(memory_space=pl.ANY),
        ),
        scratch_shapes=[
            pltpu.MemorySpace.VMEM((R, 2048), jnp.float32),
            pltpu.MemorySpace.VMEM((2, SUB, R, N_HALF), jnp.bfloat16),
            pltpu.MemorySpace.VMEM((R, N_HALF), jnp.bfloat16),
            pltpu.MemorySpace.VMEM((R, N_HALF), jnp.float32),
            pltpu.SemaphoreType.DMA((3,)),
            pltpu.SemaphoreType.DMA((2, N_DEV - 1, SUB)),
            pltpu.SemaphoreType.DMA((2, N_DEV - 1, SUB)),
        ],
        compiler_params=pltpu.CompilerParams(
            collective_id=0,
            vmem_limit_bytes=60 * 1024 * 1024,
        ),
    )(x, wb)
    return out
